# Initial kernel scaffold; baseline (speedup 1.0000x reference)
import jax
import jax.numpy as jnp
from jax.experimental import pallas as pl


def kernel(x):
    raise NotImplementedError("write your pallas kernel here")


# speedup vs baseline: 26.8548x; 26.8548x over previous
import jax
import jax.numpy as jnp
from jax.experimental import pallas as pl

_B = 8
_D = 3
_N = 4096
_R = 512


def _knn_mean_body(x_ref, out_ref):
    rb = pl.program_id(1)
    X = x_ref[0]
    rows = x_ref[0, :, pl.ds(rb * _R, _R)]
    xx = jnp.sum(X * X, axis=0, keepdims=True)
    xx_r = jnp.sum(rows * rows, axis=0)[:, None]
    G = jax.lax.dot_general(
        rows, X, dimension_numbers=(((0,), (0,)), ((), ())),
        precision=jax.lax.Precision.DEFAULT)
    P = 2.0 * G - xx_r - xx

    iota = jax.lax.broadcasted_iota(jnp.int32, (_R, _N), 1)
    W = jnp.zeros((_R, _N), jnp.float32)
    for t in range(5):
        idx = jnp.argmax(P, axis=1)
        onehot = iota == idx[:, None]
        if t > 0:
            W += onehot.astype(jnp.float32)
        P = jnp.where(onehot, -jnp.inf, P)

    M = jax.lax.dot_general(
        X, W, dimension_numbers=(((1,), (1,)), ((), ())),
        precision=jax.lax.Precision.HIGHEST)
    out_ref[0] = M * 0.25


def _neighbor_means(x):
    return pl.pallas_call(
        _knn_mean_body,
        grid=(_B, _N // _R),
        in_specs=[pl.BlockSpec((1, _D, _N), lambda b, r: (b, 0, 0))],
        out_specs=pl.BlockSpec((1, _D, _R), lambda b, r: (b, 0, r)),
        out_shape=jax.ShapeDtypeStruct((_B, _D, _N), jnp.float32),
    )(x)


def kernel(x):
    means = _neighbor_means(x)
    return jnp.concatenate([x, means], axis=2)


# speedup vs baseline: 28.1302x; 1.0475x over previous
import numpy as np

import jax
import jax.numpy as jnp
from jax.experimental import pallas as pl
from jax.sharding import Mesh, PartitionSpec

try:
    _shard_map = jax.shard_map
except AttributeError:
    from jax.experimental.shard_map import shard_map as _shard_map

_D = 3
_N = 4096
_R = 512


def _knn_mean_body(x_ref, out_ref):
    rb = pl.program_id(1)
    X = x_ref[0]
    rows = x_ref[0, :, pl.ds(rb * _R, _R)]
    xx = jnp.sum(X * X, axis=0, keepdims=True)
    xx_r = jnp.sum(rows * rows, axis=0)[:, None]
    G = jax.lax.dot_general(
        rows, X, dimension_numbers=(((0,), (0,)), ((), ())),
        precision=jax.lax.Precision.DEFAULT)
    P = 2.0 * G - xx_r - xx

    iota = jax.lax.broadcasted_iota(jnp.int32, (_R, _N), 1)
    W = jnp.zeros((_R, _N), jnp.float32)
    for t in range(5):
        idx = jnp.argmax(P, axis=1)
        onehot = iota == idx[:, None]
        if t > 0:
            W += onehot.astype(jnp.float32)
        P = jnp.where(onehot, -jnp.inf, P)

    M = jax.lax.dot_general(
        X, W, dimension_numbers=(((1,), (1,)), ((), ())),
        precision=jax.lax.Precision.HIGHEST)
    out_ref[0] = M * 0.25


def _neighbor_means(x):
    b = x.shape[0]
    return pl.pallas_call(
        _knn_mean_body,
        grid=(b, _N // _R),
        in_specs=[pl.BlockSpec((1, _D, _N), lambda b, r: (b, 0, 0))],
        out_specs=pl.BlockSpec((1, _D, _R), lambda b, r: (b, 0, r)),
        out_shape=jax.ShapeDtypeStruct((b, _D, _N), jnp.float32),
    )(x)


def kernel(x):
    devs = [d for d in jax.devices() if d.platform == "tpu"]
    n_shard = 2 if len(devs) >= 2 and x.shape[0] % 2 == 0 else 1
    if n_shard > 1:
        mesh = Mesh(np.array(devs[:n_shard]), ("d",))
        means = _shard_map(
            _neighbor_means, mesh=mesh,
            in_specs=PartitionSpec("d"), out_specs=PartitionSpec("d"),
            check_vma=False,
        )(x)
    else:
        means = _neighbor_means(x)
    return jnp.concatenate([x, means], axis=2)
